# Initial kernel scaffold; baseline (speedup 1.0000x reference)
import jax
import jax.numpy as jnp
from jax.experimental import pallas as pl


def kernel(x, w_cd, b_cd, w_cm, b_cm, w_d1, b_d1, g_d1, be_d1, w_d2, b_d2, g_d2, be_d2, w_d3, b_d3, w_m1, b_m1, g_m1, be_m1, w_m2, b_m2, g_m2, be_m2, w_m3, b_m3, w_g1, b_g1, g_g1, be_g1, w_g2, b_g2):
    raise NotImplementedError("write your pallas kernel here")


# speedup vs baseline: 10.0535x; 10.0535x over previous
import functools
import math

import jax
import jax.numpy as jnp
from jax import lax
from jax.experimental import pallas as pl
from jax.experimental.pallas import tpu as pltpu

_K = 10
_GROUPS = 4
_EPS = 1e-5
_GELU_C = math.sqrt(2.0 / math.pi)


def _gelu(x):
    return 0.5 * x * (1.0 + jnp.tanh(_GELU_C * (x + 0.044715 * x ** 3)))


def _dot(a, b, precision=lax.Precision.DEFAULT):
    return lax.dot_general(a, b, (((1,), (0,)), ((), ())),
                           preferred_element_type=jnp.float32,
                           precision=precision)


def _gn_gelu(h, gamma, beta):
    C, N = h.shape
    gs = C // _GROUPS
    outs = []
    for g in range(_GROUPS):
        seg = lax.slice(h, (g * gs, 0), ((g + 1) * gs, N))
        mu = jnp.mean(seg)
        cen = seg - mu
        var = jnp.mean(cen * cen)
        outs.append(cen * lax.rsqrt(var + _EPS))
    hn = jnp.concatenate(outs, axis=0)
    return _gelu(hn * gamma + beta)


def _nbr_mean(xt3, coords, n, r):
    xx = jnp.sum(xt3 * xt3, axis=1, keepdims=True)
    accs = []
    for t in range(n // r):
        cr = lax.slice(coords, (0, t * r), (3, (t + 1) * r))
        xxr = jnp.sum(cr * cr, axis=0, keepdims=True)
        inner = jnp.float32(-2.0) * _dot(xt3, cr)
        key = (-xxr) - inner - xx
        iota = lax.broadcasted_iota(jnp.int32, (n, r), 0)

        def body(i, carry):
            key, acc = carry
            m = jnp.max(key, axis=0, keepdims=True)
            eq = key == m
            idx = jnp.min(jnp.where(eq, iota, n), axis=0, keepdims=True)
            oh = (iota == idx).astype(jnp.float32)
            acc = acc + _dot(coords, oh)
            key = key - oh * jnp.float32(1e30)
            return key, acc

        _, acc = lax.fori_loop(0, _K, body,
                               (key, jnp.zeros((3, r), jnp.float32)))
        accs.append(acc)
    return jnp.concatenate(accs, axis=1) * jnp.float32(1.0 / _K)


def _edge_head(nm, c3, w_c, b_c):
    wa = lax.slice(w_c, (0, 0), (w_c.shape[0], 3))
    wb = lax.slice(w_c, (0, 3), (w_c.shape[0], 6))
    return _dot(wa, nm) + _dot(wb - wa, c3) + b_c


def _mlp3(h, w1, b1, g1, be1, w2, b2, g2, be2, w3, b3):
    h = _gn_gelu(_dot(w1, h) + b1, g1, be1)
    h = _gn_gelu(_dot(w2, h) + b2, g2, be2)
    return _dot(w3, h) + b3


def _fused_kernel(xt_ref, x_ref, wcd_ref, bcd_ref, wcm_ref, bcm_ref,
                  wd1_ref, bd1_ref, gd1_ref, bed1_ref,
                  wd2_ref, bd2_ref, gd2_ref, bed2_ref, wd3_ref, bd3_ref,
                  wm1_ref, bm1_ref, gm1_ref, bem1_ref,
                  wm2_ref, bm2_ref, gm2_ref, bem2_ref, wm3_ref, bm3_ref,
                  wg1_ref, bg1_ref, gg1_ref, beg1_ref, wg2_ref, bg2_ref,
                  out_ref, *, n, r):
    xt = xt_ref[0]
    x6 = x_ref[0]
    xtd = lax.slice(xt, (0, 0), (n, 3))
    xtm = lax.slice(xt, (0, 3), (n, 6))
    cd = lax.slice(x6, (0, 0), (3, n))
    cm = lax.slice(x6, (3, 0), (6, n))

    nmd = _nbr_mean(xtd, cd, n, r)
    nmm = _nbr_mean(xtm, cm, n, r)

    xd = _edge_head(nmd, cd, wcd_ref[...], bcd_ref[...])
    xm = _edge_head(nmm, cm, wcm_ref[...], bcm_ref[...])

    xdo = _mlp3(xd, wd1_ref[...], bd1_ref[...], gd1_ref[...], bed1_ref[...],
                wd2_ref[...], bd2_ref[...], gd2_ref[...], bed2_ref[...],
                wd3_ref[...], bd3_ref[...])
    xmo = _mlp3(xm, wm1_ref[...], bm1_ref[...], gm1_ref[...], bem1_ref[...],
                wm2_ref[...], bm2_ref[...], gm2_ref[...], bem2_ref[...],
                wm3_ref[...], bm3_ref[...])

    xc = jnp.concatenate([xdo, xmo], axis=0)
    h = _gn_gelu(_dot(wg1_ref[...], xc) + bg1_ref[...],
                 gg1_ref[...], beg1_ref[...])
    out_ref[0] = _dot(wg2_ref[...], h) + bg2_ref[...]


def kernel(x, w_cd, b_cd, w_cm, b_cm, w_d1, b_d1, g_d1, be_d1, w_d2, b_d2,
           g_d2, be_d2, w_d3, b_d3, w_m1, b_m1, g_m1, be_m1, w_m2, b_m2,
           g_m2, be_m2, w_m3, b_m3, w_g1, b_g1, g_g1, be_g1, w_g2, b_g2):
    B, C, N = x.shape
    R = min(512, N)
    xt = jnp.transpose(x, (0, 2, 1))

    def col(v):
        return v.reshape(-1, 1)

    weights = (w_cd, col(b_cd), w_cm, col(b_cm),
               w_d1, col(b_d1), col(g_d1), col(be_d1),
               w_d2, col(b_d2), col(g_d2), col(be_d2), w_d3, col(b_d3),
               w_m1, col(b_m1), col(g_m1), col(be_m1),
               w_m2, col(b_m2), col(g_m2), col(be_m2), w_m3, col(b_m3),
               w_g1, col(b_g1), col(g_g1), col(be_g1), w_g2, col(b_g2))

    def wspec(v):
        nd = v.ndim
        return pl.BlockSpec(v.shape, lambda b, _nd=nd: (0,) * _nd)

    in_specs = [
        pl.BlockSpec((1, N, C), lambda b: (b, 0, 0)),
        pl.BlockSpec((1, C, N), lambda b: (b, 0, 0)),
    ] + [wspec(v) for v in weights]

    out = pl.pallas_call(
        functools.partial(_fused_kernel, n=N, r=R),
        grid=(B,),
        in_specs=in_specs,
        out_specs=pl.BlockSpec((1, 512, N), lambda b: (b, 0, 0)),
        out_shape=jax.ShapeDtypeStruct((B, 512, N), jnp.float32),
        compiler_params=pltpu.CompilerParams(
            dimension_semantics=("arbitrary",)),
    )(xt, x, *weights)
    return out


# speedup vs baseline: 15.1292x; 1.5049x over previous
import functools
import math

import jax
import jax.numpy as jnp
from jax import lax
from jax.experimental import pallas as pl
from jax.experimental.pallas import tpu as pltpu

_K = 10
_GROUPS = 4
_EPS = 1e-5
_GELU_C = math.sqrt(2.0 / math.pi)


def _gelu(x):
    return 0.5 * x * (1.0 + jnp.tanh(_GELU_C * (x + 0.044715 * x ** 3)))


def _dot(a, b, precision=lax.Precision.DEFAULT):
    return lax.dot_general(a, b, (((1,), (0,)), ((), ())),
                           preferred_element_type=jnp.float32,
                           precision=precision)


def _gn_gelu(h, gamma, beta):
    C, N = h.shape
    gs = C // _GROUPS
    outs = []
    for g in range(_GROUPS):
        seg = lax.slice(h, (g * gs, 0), ((g + 1) * gs, N))
        mu = jnp.mean(seg)
        cen = seg - mu
        var = jnp.mean(cen * cen)
        outs.append(cen * lax.rsqrt(var + _EPS))
    hn = jnp.concatenate(outs, axis=0)
    return _gelu(hn * gamma + beta)


def _nbr_mean(xt3, coords, n, r):
    xx = jnp.sum(xt3 * xt3, axis=1, keepdims=True)
    accs = []
    for t in range(n // r):
        cr = lax.slice(coords, (0, t * r), (3, (t + 1) * r))
        xxr = jnp.sum(cr * cr, axis=0, keepdims=True)
        inner = jnp.float32(-2.0) * _dot(xt3, cr)
        key = (-xxr) - inner - xx

        def body(i, carry):
            key, acc = carry
            m = jnp.max(key, axis=0, keepdims=True)
            eq = key == m
            oh = jnp.where(eq, jnp.float32(1.0), jnp.float32(0.0))
            acc = acc + _dot(coords, oh)
            key = jnp.where(eq, jnp.float32(-3e38), key)
            return key, acc

        _, acc = lax.fori_loop(0, _K, body,
                               (key, jnp.zeros((3, r), jnp.float32)))
        accs.append(acc)
    return jnp.concatenate(accs, axis=1) * jnp.float32(1.0 / _K)


def _edge_head(nm, c3, w_c, b_c):
    wa = lax.slice(w_c, (0, 0), (w_c.shape[0], 3))
    wb = lax.slice(w_c, (0, 3), (w_c.shape[0], 6))
    return _dot(wa, nm) + _dot(wb - wa, c3) + b_c


def _mlp3(h, w1, b1, g1, be1, w2, b2, g2, be2, w3, b3):
    h = _gn_gelu(_dot(w1, h) + b1, g1, be1)
    h = _gn_gelu(_dot(w2, h) + b2, g2, be2)
    return _dot(w3, h) + b3


def _fused_kernel(xt_ref, x_ref, wcd_ref, bcd_ref, wcm_ref, bcm_ref,
                  wd1_ref, bd1_ref, gd1_ref, bed1_ref,
                  wd2_ref, bd2_ref, gd2_ref, bed2_ref, wd3_ref, bd3_ref,
                  wm1_ref, bm1_ref, gm1_ref, bem1_ref,
                  wm2_ref, bm2_ref, gm2_ref, bem2_ref, wm3_ref, bm3_ref,
                  wg1_ref, bg1_ref, gg1_ref, beg1_ref, wg2_ref, bg2_ref,
                  out_ref, *, n, r):
    xt = xt_ref[0]
    x6 = x_ref[0]
    xtd = lax.slice(xt, (0, 0), (n, 3))
    xtm = lax.slice(xt, (0, 3), (n, 6))
    cd = lax.slice(x6, (0, 0), (3, n))
    cm = lax.slice(x6, (3, 0), (6, n))

    nmd = _nbr_mean(xtd, cd, n, r)
    nmm = _nbr_mean(xtm, cm, n, r)

    xd = _edge_head(nmd, cd, wcd_ref[...], bcd_ref[...])
    xm = _edge_head(nmm, cm, wcm_ref[...], bcm_ref[...])

    xdo = _mlp3(xd, wd1_ref[...], bd1_ref[...], gd1_ref[...], bed1_ref[...],
                wd2_ref[...], bd2_ref[...], gd2_ref[...], bed2_ref[...],
                wd3_ref[...], bd3_ref[...])
    xmo = _mlp3(xm, wm1_ref[...], bm1_ref[...], gm1_ref[...], bem1_ref[...],
                wm2_ref[...], bm2_ref[...], gm2_ref[...], bem2_ref[...],
                wm3_ref[...], bm3_ref[...])

    xc = jnp.concatenate([xdo, xmo], axis=0)
    h = _gn_gelu(_dot(wg1_ref[...], xc) + bg1_ref[...],
                 gg1_ref[...], beg1_ref[...])
    out_ref[0] = _dot(wg2_ref[...], h) + bg2_ref[...]


def kernel(x, w_cd, b_cd, w_cm, b_cm, w_d1, b_d1, g_d1, be_d1, w_d2, b_d2,
           g_d2, be_d2, w_d3, b_d3, w_m1, b_m1, g_m1, be_m1, w_m2, b_m2,
           g_m2, be_m2, w_m3, b_m3, w_g1, b_g1, g_g1, be_g1, w_g2, b_g2):
    B, C, N = x.shape
    R = min(512, N)
    xt = jnp.transpose(x, (0, 2, 1))

    def col(v):
        return v.reshape(-1, 1)

    weights = (w_cd, col(b_cd), w_cm, col(b_cm),
               w_d1, col(b_d1), col(g_d1), col(be_d1),
               w_d2, col(b_d2), col(g_d2), col(be_d2), w_d3, col(b_d3),
               w_m1, col(b_m1), col(g_m1), col(be_m1),
               w_m2, col(b_m2), col(g_m2), col(be_m2), w_m3, col(b_m3),
               w_g1, col(b_g1), col(g_g1), col(be_g1), w_g2, col(b_g2))

    def wspec(v):
        nd = v.ndim
        return pl.BlockSpec(v.shape, lambda b, _nd=nd: (0,) * _nd)

    in_specs = [
        pl.BlockSpec((1, N, C), lambda b: (b, 0, 0)),
        pl.BlockSpec((1, C, N), lambda b: (b, 0, 0)),
    ] + [wspec(v) for v in weights]

    out = pl.pallas_call(
        functools.partial(_fused_kernel, n=N, r=R),
        grid=(B,),
        in_specs=in_specs,
        out_specs=pl.BlockSpec((1, 512, N), lambda b: (b, 0, 0)),
        out_shape=jax.ShapeDtypeStruct((B, 512, N), jnp.float32),
        compiler_params=pltpu.CompilerParams(
            dimension_semantics=("arbitrary",)),
    )(xt, x, *weights)
    return out
